# baseline (speedup 1.0000x reference)
import functools

import jax
import jax.numpy as jnp
from jax import lax
from jax.experimental import pallas as pl
from jax.experimental.pallas import tpu as pltpu
from jax.experimental.pallas import tpu_sc as plsc

N = 100000
E = 1600000
W = 2000000
T = 6

C = 320
NCHUNK = W // C
NC = 2
NS = 16
NW = NC * NS
NLANES = 3
SLOTS = NLANES * NW
TRIPS = 2 * (-(-NCHUNK // (2 * SLOTS)))
G = C // 16
UNROLL = 4

_mesh = plsc.VectorSubcoreMesh(core_axis_name="c", subcore_axis_name="s")


def _scratch():
    types = [pltpu.VMEM((N + 1,), jnp.int32)]
    for _ in range(NLANES):
        types += [
            pltpu.VMEM(((T - 1) * C,), jnp.float32),
            pltpu.VMEM((C,), jnp.int32),
            pltpu.VMEM(((T - 1) * C,), jnp.float32),
            pltpu.VMEM((C,), jnp.int32),
            pltpu.VMEM((C * T,), jnp.float32),
            pltpu.VMEM((C,), jnp.int32),
            pltpu.VMEM((C,), jnp.int32),
            pltpu.VMEM((C,), jnp.int32),
            pltpu.VMEM((C,), jnp.int32),
        ]
    types += [pltpu.SemaphoreType.DMA] * NLANES
    types += [pltpu.SemaphoreType.DMA]
    return types


@functools.partial(
    pl.kernel,
    out_type=jax.ShapeDtypeStruct((W * T,), jnp.float32),
    mesh=_mesh,
    compiler_params=pltpu.CompilerParams(needs_layout_passes=False),
    scratch_types=_scratch(),
)
def _walk(indptr_hbm, indices_hbm, start_hbm, rand_hbm, out_hbm,
          indptr_v, *scr):
    lanes = [scr[9 * l:9 * (l + 1)] for l in range(NLANES)]
    sems = scr[9 * NLANES:9 * NLANES + NLANES]
    semIn = scr[9 * NLANES + NLANES]
    wid = lax.axis_index("s") * NC + lax.axis_index("c")
    pltpu.sync_copy(indptr_hbm, indptr_v)

    def base(i, l):
        return jnp.minimum(wid + l * NW + i * SLOTS, NCHUNK - 1) * C

    def input_copies(i, l, p, make):
        rand0, start0, rand1, start1 = lanes[l][:4]
        rand_v, start_v = (rand0, start0) if p == 0 else (rand1, start1)
        b = base(i, l)
        mk = pltpu.make_async_copy if make else pltpu.async_copy
        return (
            mk(start_hbm.at[pl.ds(b, C)], start_v, semIn),
            mk(rand_hbm.at[pl.ds(b * T, (T - 1) * C)], rand_v, semIn),
        )

    def fire_inputs(i, p):
        for l in range(NLANES):
            input_copies(i, l, p, make=False)

    def wait_inputs(i, l, p):
        for cp in input_copies(i, l, p, make=True):
            cp.wait()

    def compute(t, l, p):
        rand0, start0, rand1, start1, out_v, curr_v, addr_v, deg_v, nbr_v = \
            lanes[l]
        rand_v, start_v = (rand0, start0) if p == 0 else (rand1, start1)
        curr_src = start_v if t <= 1 else curr_v

        @plsc.parallel_loop(0, G, 1, unroll=UNROLL)
        def body(g):
            sl = pl.ds(g * 16, 16)
            tsl = pl.ds(t * C + g * 16, 16)
            curr = curr_src[sl]
            if t > 0:
                curr = jnp.where(deg_v[sl] > 0, nbr_v[sl], curr)
                curr_v[sl] = curr
            out_v[tsl] = curr.astype(jnp.float32)
            if t < T - 1:
                rs = plsc.load_gather(indptr_v, [curr])
                re = plsc.load_gather(indptr_v, [curr + 1])
                deg = re - rs
                u = rand_v[tsl]
                off = (u * deg.astype(jnp.float32)).astype(jnp.int32)
                off = jnp.minimum(off, jnp.maximum(deg - 1, 0))
                addr_v[sl] = rs + off
                deg_v[sl] = deg

    def fire_gather(l):
        addr_v, nbr_v = lanes[l][6], lanes[l][8]
        pltpu.async_copy(indices_hbm.at[addr_v], nbr_v, sems[l])

    def wait_gather(l):
        addr_v, nbr_v = lanes[l][6], lanes[l][8]
        pltpu.make_async_copy(indices_hbm.at[addr_v], nbr_v, sems[l]).wait()

    def fire_out(i, l):
        out_v = lanes[l][4]
        pltpu.async_copy(out_v, out_hbm.at[pl.ds(base(i, l) * T, C * T)],
                         sems[l])

    def drain_out(i):
        for l in range(NLANES):
            out_v = lanes[l][4]
            pltpu.make_async_copy(
                out_v, out_hbm.at[pl.ds(base(i, l) * T, C * T)],
                sems[l]).wait()

    def trip(i, p, fire_next):
        wait_inputs(i, 0, p)

        @pl.when(fire_next)
        def _():
            fire_inputs(i + 1, 1 - p)

        @pl.when(i > 0)
        def _():
            drain_out(i - 1)
        for t in range(T):
            for l in range(NLANES):
                if t > 0:
                    wait_gather(l)
                elif l > 0:
                    wait_inputs(i, l, p)
                compute(t, l, p)
                if t < T - 1:
                    fire_gather(l)
        for l in range(NLANES):
            fire_out(i, l)

    fire_inputs(0, 0)
    jmax = TRIPS // 2

    def two_trips(j, _):
        trip(2 * j, 0, fire_next=jnp.bool_(True))
        trip(2 * j + 1, 1, fire_next=(j < jmax - 1))
        return 0

    lax.fori_loop(0, jmax, two_trips, 0)
    drain_out(TRIPS - 1)


def kernel(indptr, indices, start_nodes, rand_u):
    rand_t = rand_u.reshape(NCHUNK, C, T).transpose(0, 2, 1).reshape(W * T)
    flat = _walk(indptr.astype(jnp.int32), indices.astype(jnp.int32),
                 start_nodes.astype(jnp.int32), rand_t)
    return flat.reshape(NCHUNK, T, C).transpose(0, 2, 1).reshape(W, T)


# speedup vs baseline: 6.5080x; 6.5080x over previous
import functools

import jax
import jax.numpy as jnp
from jax import lax
from jax.experimental import pallas as pl
from jax.experimental.pallas import tpu as pltpu
from jax.experimental.pallas import tpu_sc as plsc

N = 100000
E = 1600000
W = 2000000
T = 6

C = 640
NCHUNK = W // C
NC = 2
NS = 16
NW = NC * NS
TRIPS = 2 * (-(-NCHUNK // (4 * NW)))
G = C // 16
UNROLL = 2

_mesh = plsc.VectorSubcoreMesh(core_axis_name="c", subcore_axis_name="s")


def _in_bank():
    return [
        pltpu.VMEM(((T - 1) * C,), jnp.float32),
        pltpu.VMEM((C,), jnp.int32),
    ]


def _lane():
    return [
        pltpu.VMEM((C * T,), jnp.float32),
        pltpu.VMEM((C,), jnp.int32),
        pltpu.VMEM((C,), jnp.int32),
        pltpu.VMEM((C,), jnp.int32),
        pltpu.VMEM((C,), jnp.int32),
    ]


@functools.partial(
    pl.kernel,
    out_type=jax.ShapeDtypeStruct((W * T,), jnp.float32),
    mesh=_mesh,
    compiler_params=pltpu.CompilerParams(needs_layout_passes=False),
    scratch_types=(
        [pltpu.VMEM((N + 1,), jnp.int32)]
        + _in_bank() + _in_bank()
        + _in_bank() + _in_bank()
        + _lane() + _lane()
        + [pltpu.SemaphoreType.DMA,
           pltpu.SemaphoreType.DMA,
           pltpu.SemaphoreType.DMA]
    ),
)
def _walk(indptr_hbm, indices_hbm, start_hbm, rand_hbm, out_hbm,
          indptr_v,
          randA0, startA0, randB0, startB0,
          randA1, startA1, randB1, startB1,
          outA, currA, addrA, degA, nbrA,
          outB, currB, addrB, degB, nbrB,
          semA, semB, semIn):
    wid = lax.axis_index("s") * NC + lax.axis_index("c")
    pltpu.sync_copy(indptr_hbm, indptr_v)
    banks = ((randA0, startA0, randB0, startB0),
             (randA1, startA1, randB1, startB1))

    def chunk_bases(i):
        cA = jnp.minimum(wid + i * (2 * NW), NCHUNK - 1)
        cB = jnp.minimum(wid + NW + i * (2 * NW), NCHUNK - 1)
        return cA * C, cB * C

    def input_copies(i, bank, make):
        randA_v, startA_v, randB_v, startB_v = bank
        baseA, baseB = chunk_bases(i)
        mk = pltpu.make_async_copy if make else pltpu.async_copy
        return (
            mk(start_hbm.at[pl.ds(baseA, C)], startA_v, semIn),
            mk(rand_hbm.at[pl.ds(baseA * T, (T - 1) * C)], randA_v, semIn),
            mk(start_hbm.at[pl.ds(baseB, C)], startB_v, semIn),
            mk(rand_hbm.at[pl.ds(baseB * T, (T - 1) * C)], randB_v, semIn),
        )

    def fire_inputs(i, bank):
        input_copies(i, bank, make=False)

    def wait_inputs_a(i, bank):
        cps = input_copies(i, bank, make=True)
        cps[0].wait()
        cps[1].wait()

    def wait_inputs_b(i, bank):
        cps = input_copies(i, bank, make=True)
        cps[2].wait()
        cps[3].wait()

    def compute(t, rand_v, out_v, curr_src, curr_v, addr_v, deg_v, nbr_v):
        @plsc.parallel_loop(0, G, 1, unroll=UNROLL)
        def body(g):
            sl = pl.ds(g * 16, 16)
            tsl = pl.ds(t * C + g * 16, 16)
            curr = curr_src[sl]
            if t > 0:
                curr = jnp.where(deg_v[sl] > 0, nbr_v[sl], curr)
                curr_v[sl] = curr
            out_v[tsl] = curr.astype(jnp.float32)
            if t < T - 1:
                rs = plsc.load_gather(indptr_v, [curr])
                re = plsc.load_gather(indptr_v, [curr + 1])
                deg = re - rs
                u = rand_v[tsl]
                off = (u * deg.astype(jnp.float32)).astype(jnp.int32)
                off = jnp.minimum(off, jnp.maximum(deg - 1, 0))
                addr_v[sl] = rs + off
                deg_v[sl] = deg

    def drain_out(i):
        baseA, baseB = chunk_bases(i)
        pltpu.make_async_copy(outA, out_hbm.at[pl.ds(baseA * T, C * T)],
                              semA).wait()
        pltpu.make_async_copy(outB, out_hbm.at[pl.ds(baseB * T, C * T)],
                              semB).wait()

    def trip(i, p, fire_next):
        randA_v, startA_v, randB_v, startB_v = banks[p]
        baseA, baseB = chunk_bases(i)
        wait_inputs_a(i, banks[p])
        if fire_next is not None:
            @pl.when(fire_next)
            def _():
                fire_inputs(i + 1, banks[1 - p])

        @pl.when(i > 0)
        def _():
            drain_out(i - 1)
        gA = gB = None
        for t in range(T):
            if t > 0:
                gA.wait()
            compute(t, randA_v, outA, startA_v if t <= 1 else currA,
                    currA, addrA, degA, nbrA)
            if t < T - 1:
                gA = pltpu.async_copy(indices_hbm.at[addrA], nbrA, semA)
            if t == 0:
                wait_inputs_b(i, banks[p])
            else:
                gB.wait()
            compute(t, randB_v, outB, startB_v if t <= 1 else currB,
                    currB, addrB, degB, nbrB)
            if t < T - 1:
                gB = pltpu.async_copy(indices_hbm.at[addrB], nbrB, semB)
        pltpu.async_copy(outA, out_hbm.at[pl.ds(baseA * T, C * T)], semA)
        pltpu.async_copy(outB, out_hbm.at[pl.ds(baseB * T, C * T)], semB)

    fire_inputs(0, banks[0])
    jmax = TRIPS // 2

    def two_trips(j, _):
        trip(2 * j, 0, fire_next=jnp.bool_(True))
        trip(2 * j + 1, 1, fire_next=(j < jmax - 1))
        return 0

    lax.fori_loop(0, jmax, two_trips, 0)
    drain_out(TRIPS - 1)


def kernel(indptr, indices, start_nodes, rand_u):
    rand_t = rand_u.reshape(NCHUNK, C, T).transpose(0, 2, 1).reshape(W * T)
    flat = _walk(indptr.astype(jnp.int32), indices.astype(jnp.int32),
                 start_nodes.astype(jnp.int32), rand_t)
    return flat.reshape(NCHUNK, T, C).transpose(0, 2, 1).reshape(W, T)
